# baseline (speedup 1.0000x reference)
import functools

import jax
import jax.numpy as jnp
from jax import lax
from jax.experimental import pallas as pl
from jax.experimental.pallas import tpu as pltpu
from jax.experimental.pallas import tpu_sc as plsc

N_REGIONS = 6
BOUND = 2.5

_B, _C, _H, _W = 8, 96, 224, 224
_N = _B * _C * _H * _W
_ROW = _H * _W
_NW = 32
_PER_W = _N // _NW
_CHUNK = _ROW // 2
_CHUNKS_PER_W = _PER_W // _CHUNK
_ROWS_PER_W = _PER_W // _ROW
_VECS = _CHUNK // 16
_TAB = _B * _C * N_REGIONS


def _sc_body(x_hbm, a_hbm, b_hbm, out_hbm, in_v, out_v, a_tab, b_tab):
    wid = lax.axis_index("s") * 2 + lax.axis_index("c")
    pltpu.sync_copy(a_hbm, a_tab)
    pltpu.sync_copy(b_hbm, b_tab)

    def chunk_body(k, _):
        off = wid * _PER_W + k * _CHUNK
        base = (wid * _ROWS_PER_W + k // 2) * N_REGIONS
        pltpu.sync_copy(x_hbm.at[pl.ds(off, _CHUNK)], in_v)

        @plsc.parallel_loop(0, _CHUNK, step=16, unroll=16)
        def vec_body(i):
            v = in_v[pl.ds(i, 16)]
            s = v * (0.5 * N_REGIONS / BOUND) + (0.5 * N_REGIONS)
            sc = jnp.minimum(jnp.maximum(s, 0.0), float(N_REGIONS) * 0.999)
            idx = sc.astype(jnp.int32) + base
            a = plsc.load_gather(a_tab, [idx])
            b = plsc.load_gather(b_tab, [idx])
            out_v[pl.ds(i, 16)] = a + b * s
        pltpu.sync_copy(out_v, out_hbm.at[pl.ds(off, _CHUNK)])
        return 0

    lax.fori_loop(0, _CHUNKS_PER_W, chunk_body, 0)


def kernel(x, points):
    B, C, H, W = x.shape

    slopes = points[:, 1:] - points[:, :-1]
    intercepts = points[:, :-1] - slopes * jnp.arange(
        N_REGIONS, dtype=points.dtype
    )[None, :]
    a_flat = jnp.tile(intercepts, (B, 1)).reshape(-1)
    b_flat = jnp.tile(slopes, (B, 1)).reshape(-1)

    xf = x.reshape(-1)
    sc_kernel = functools.partial(
        pl.kernel,
        out_type=jax.ShapeDtypeStruct((_N,), jnp.float32),
        mesh=plsc.VectorSubcoreMesh(core_axis_name="c", subcore_axis_name="s"),
        compiler_params=pltpu.CompilerParams(needs_layout_passes=False),
        scratch_types=[
            pltpu.VMEM((_CHUNK,), jnp.float32),
            pltpu.VMEM((_CHUNK,), jnp.float32),
            pltpu.VMEM((_TAB,), jnp.float32),
            pltpu.VMEM((_TAB,), jnp.float32),
        ],
    )(_sc_body)
    out = sc_kernel(xf, a_flat, b_flat)
    return out.reshape(B, C, H, W)


# speedup vs baseline: 1.1558x; 1.1558x over previous
import functools

import jax
import jax.numpy as jnp
from jax import lax
from jax.experimental import pallas as pl
from jax.experimental.pallas import tpu as pltpu
from jax.experimental.pallas import tpu_sc as plsc

N_REGIONS = 6
BOUND = 2.5

_B, _C, _H, _W = 8, 96, 224, 224
_N = _B * _C * _H * _W
_ROW = _H * _W
_NW = 32
_PER_W = _N // _NW
_CHUNK = _ROW
_CHUNKS_PER_W = _PER_W // _CHUNK
_ROWS_PER_W = _PER_W // _ROW
_VECS = _CHUNK // 16
_TAB = _B * _C * N_REGIONS


def _sc_body(x_hbm, a_hbm, b_hbm, out_hbm, in_v, out_v, a_tab, b_tab):
    wid = lax.axis_index("s") * 2 + lax.axis_index("c")
    pltpu.sync_copy(a_hbm, a_tab)
    pltpu.sync_copy(b_hbm, b_tab)

    def chunk_body(k, _):
        off = wid * _PER_W + k * _CHUNK
        base = (wid * _ROWS_PER_W + k) * N_REGIONS
        pltpu.sync_copy(x_hbm.at[pl.ds(off, _CHUNK)], in_v)

        @plsc.parallel_loop(0, _CHUNK, step=16, unroll=8)
        def vec_body(i):
            v = in_v[pl.ds(i, 16)]
            s = v * (0.5 * N_REGIONS / BOUND) + (0.5 * N_REGIONS)
            sc = jnp.minimum(jnp.maximum(s, 0.0), float(N_REGIONS) * 0.999)
            idx = sc.astype(jnp.int32) + base
            a = plsc.load_gather(a_tab, [idx])
            b = plsc.load_gather(b_tab, [idx])
            out_v[pl.ds(i, 16)] = a + b * s
        pltpu.sync_copy(out_v, out_hbm.at[pl.ds(off, _CHUNK)])
        return 0

    lax.fori_loop(0, _CHUNKS_PER_W, chunk_body, 0)


def kernel(x, points):
    B, C, H, W = x.shape

    slopes = points[:, 1:] - points[:, :-1]
    intercepts = points[:, :-1] - slopes * jnp.arange(
        N_REGIONS, dtype=points.dtype
    )[None, :]
    a_flat = jnp.tile(intercepts, (B, 1)).reshape(-1)
    b_flat = jnp.tile(slopes, (B, 1)).reshape(-1)

    xf = x.reshape(-1)
    sc_kernel = functools.partial(
        pl.kernel,
        out_type=jax.ShapeDtypeStruct((_N,), jnp.float32),
        mesh=plsc.VectorSubcoreMesh(core_axis_name="c", subcore_axis_name="s"),
        compiler_params=pltpu.CompilerParams(needs_layout_passes=False),
        scratch_types=[
            pltpu.VMEM((_CHUNK,), jnp.float32),
            pltpu.VMEM((_CHUNK,), jnp.float32),
            pltpu.VMEM((_TAB,), jnp.float32),
            pltpu.VMEM((_TAB,), jnp.float32),
        ],
    )(_sc_body)
    out = sc_kernel(xf, a_flat, b_flat)
    return out.reshape(B, C, H, W)


# speedup vs baseline: 1.4394x; 1.2454x over previous
import functools

import jax
import jax.numpy as jnp
from jax import lax
from jax.experimental import pallas as pl
from jax.experimental.pallas import tpu as pltpu
from jax.experimental.pallas import tpu_sc as plsc

N_REGIONS = 6
BOUND = 2.5

_B, _C, _H, _W = 8, 96, 224, 224
_B_TC = 6
_B_SC = _B - _B_TC
_ROW = _H * _W
_NW = 32
_N_SC = _B_SC * _C * _ROW
_ROWS_PER_W = _B_SC * _C // _NW
_PER_W = _ROWS_PER_W * _ROW
_CHUNK = _ROW
_TAB = _B_SC * _C * N_REGIONS


def _pwlu_tc_kernel(x_ref, a_ref, b_ref, out_ref):
    x = x_ref[...]
    s = x * (0.5 * N_REGIONS / BOUND) + (0.5 * N_REGIONS)
    a = jnp.full_like(s, a_ref[0, 0, 0])
    b = jnp.full_like(s, b_ref[0, 0, 0])
    for j in range(1, N_REGIONS):
        m = s >= float(j)
        a = jnp.where(m, a_ref[0, 0, j], a)
        b = jnp.where(m, b_ref[0, 0, j], b)
    out_ref[...] = a + b * s


def _sc_body(x_hbm, a_hbm, b_hbm, out_hbm, in_v, out_v, a_tab, b_tab):
    wid = lax.axis_index("s") * 2 + lax.axis_index("c")
    pltpu.sync_copy(a_hbm, a_tab)
    pltpu.sync_copy(b_hbm, b_tab)

    def chunk_body(k, _):
        off = wid * _PER_W + k * _CHUNK
        base = (wid * _ROWS_PER_W + k) * N_REGIONS
        pltpu.sync_copy(x_hbm.at[pl.ds(off, _CHUNK)], in_v)

        @plsc.parallel_loop(0, _CHUNK, step=16, unroll=8)
        def vec_body(i):
            v = in_v[pl.ds(i, 16)]
            s = v * (0.5 * N_REGIONS / BOUND) + (0.5 * N_REGIONS)
            sc = jnp.minimum(jnp.maximum(s, 0.0), float(N_REGIONS) * 0.999)
            idx = sc.astype(jnp.int32) + base
            a = plsc.load_gather(a_tab, [idx])
            b = plsc.load_gather(b_tab, [idx])
            out_v[pl.ds(i, 16)] = a + b * s

        pltpu.sync_copy(out_v, out_hbm.at[pl.ds(off, _CHUNK)])
        return 0

    lax.fori_loop(0, _ROWS_PER_W, chunk_body, 0)


def kernel(x, points):
    B, C, H, W = x.shape

    slopes = points[:, 1:] - points[:, :-1]
    intercepts = points[:, :-1] - slopes * jnp.arange(
        N_REGIONS, dtype=points.dtype
    )[None, :]

    a_t = intercepts.reshape(C, 1, N_REGIONS)
    b_t = slopes.reshape(C, 1, N_REGIONS)
    x_tc = lax.slice_in_dim(x, 0, _B_TC, axis=0)
    out_tc = pl.pallas_call(
        _pwlu_tc_kernel,
        grid=(C,),
        in_specs=[
            pl.BlockSpec((_B_TC, 1, H, W), lambda c: (0, c, 0, 0)),
            pl.BlockSpec((1, 1, N_REGIONS), lambda c: (c, 0, 0)),
            pl.BlockSpec((1, 1, N_REGIONS), lambda c: (c, 0, 0)),
        ],
        out_specs=pl.BlockSpec((_B_TC, 1, H, W), lambda c: (0, c, 0, 0)),
        out_shape=jax.ShapeDtypeStruct((_B_TC, C, H, W), x.dtype),
    )(x_tc, a_t, b_t)

    a_flat = jnp.tile(intercepts, (_B_SC, 1)).reshape(-1)
    b_flat = jnp.tile(slopes, (_B_SC, 1)).reshape(-1)
    x_sc = lax.slice_in_dim(x, _B_TC, B, axis=0).reshape(-1)
    sc_kernel = functools.partial(
        pl.kernel,
        out_type=jax.ShapeDtypeStruct((_N_SC,), jnp.float32),
        mesh=plsc.VectorSubcoreMesh(core_axis_name="c", subcore_axis_name="s"),
        compiler_params=pltpu.CompilerParams(needs_layout_passes=False),
        scratch_types=[
            pltpu.VMEM((_CHUNK,), jnp.float32),
            pltpu.VMEM((_CHUNK,), jnp.float32),
            pltpu.VMEM((_TAB,), jnp.float32),
            pltpu.VMEM((_TAB,), jnp.float32),
        ],
    )(_sc_body)
    out_sc = sc_kernel(x_sc, a_flat, b_flat).reshape(_B_SC, C, H, W)

    return lax.concatenate([out_tc, out_sc], dimension=0)
